# baseline (speedup 1.0000x reference)
import functools

import jax
import jax.numpy as jnp
from jax import lax
from jax.experimental import pallas as pl
from jax.experimental.pallas import tpu as pltpu
from jax.experimental.pallas import tpu_sc as plsc

_F = 256
_D = 64
_PAIRS_PER_W = 8


def _sc_body(Bs, HWD, row_hbm, col_hbm, dep_hbm, out_hbm,
             buf_a, buf_b, rbuf, cbuf, sem_a, sem_b):
    wid = lax.axis_index("s") * 2 + lax.axis_index("c")
    q0 = wid * _PAIRS_PER_W

    bufs = (buf_a, buf_b)
    sems = (sem_a, sem_b)
    for buf in bufs:
        pltpu.sync_copy(dep_hbm, buf.at[:, pl.ds(2 * _F, _F)])

    pending = [[], []]
    for t in range(_PAIRS_PER_W):
        slot = t % 2
        buf, sem = bufs[slot], sems[slot]
        for dsc in pending[slot]:
            dsc.wait()
        pending[slot] = []

        q = q0 + t
        i = q // 16
        j = lax.rem(q, 16)
        pltpu.sync_copy(row_hbm.at[i], rbuf)
        pltpu.sync_copy(col_hbm.at[j], cbuf)

        rv = [rbuf[pl.ds(c * 16, 16)] for c in range(_F // 16)]
        cv = [cbuf[pl.ds(c * 16, 16)] for c in range(_F // 16)]

        def fill_row(r, carry):
            for c in range(_F // 16):
                buf[r, pl.ds(c * 16, 16)] = rv[c]
                buf[r, pl.ds(_F + c * 16, 16)] = cv[c]
            return carry

        lax.fori_loop(0, _D, fill_row, 0)

        base = i * 1024 + j * _D
        for b in range(Bs):
            dst = out_hbm.at[pl.ds(base + b * HWD, _D)]
            pending[slot].append(pltpu.async_copy(buf, dst, sem))

    for slot in (0, 1):
        for dsc in pending[slot]:
            dsc.wait()


def kernel(B, h, w, d, x, row_embed, col_embed, dep_embed):
    H, F = row_embed.shape
    W = col_embed.shape[0]
    D = dep_embed.shape[0]
    Bs = x.shape[0]
    HWD = H * W * D
    mesh = plsc.VectorSubcoreMesh(core_axis_name="c", subcore_axis_name="s")
    sc_call = functools.partial(
        pl.kernel,
        mesh=mesh,
        out_type=jax.ShapeDtypeStruct((Bs * HWD, 3 * F), jnp.float32),
        scratch_types=[
            pltpu.VMEM((D, 3 * F), jnp.float32),
            pltpu.VMEM((D, 3 * F), jnp.float32),
            pltpu.VMEM((F,), jnp.float32),
            pltpu.VMEM((F,), jnp.float32),
            pltpu.SemaphoreType.DMA,
            pltpu.SemaphoreType.DMA,
        ],
    )(functools.partial(_sc_body, Bs, HWD))
    out = sc_call(row_embed, col_embed, dep_embed)
    return out.reshape(Bs, HWD, 3 * F)
